# baseline (speedup 1.0000x reference)
import functools

import jax
import jax.numpy as jnp
from jax import lax
from jax.experimental import pallas as pl
from jax.experimental.pallas import tpu as pltpu
from jax.experimental.pallas import tpu_sc as plsc

B = 4096
L = 200
F = 64
NC = 2
NS = 16
NW = NC * NS
ROWS_PER_TILE = B // NW
IDX_PER_TILE = ROWS_PER_TILE * L
LANES = 16
FCHUNKS = F // LANES
CHUNK_A = 128
CHUNK_B = L - CHUNK_A


def _sc_pool_kernel(idx_hbm, table_hbm, out_hbm, idx_v, rows_v, acc_v, sem_a, sem_b):
  wid = lax.axis_index("s") * NC + lax.axis_index("c")
  base = wid * ROWS_PER_TILE

  pltpu.sync_copy(idx_hbm.at[pl.ds(base * L, IDX_PER_TILE)], idx_v)

  def start_gather(r, buf):
    off = pl.multiple_of(r * L, 8)
    da = pltpu.async_copy(
        table_hbm.at[idx_v.at[pl.ds(off, CHUNK_A)]],
        rows_v.at[buf, pl.ds(0, CHUNK_A)], sem_a)
    db = pltpu.async_copy(
        table_hbm.at[idx_v.at[pl.ds(off + CHUNK_A, CHUNK_B)]],
        rows_v.at[buf, pl.ds(CHUNK_A, CHUNK_B)], sem_b)
    return da, db

  def accum_row(r, buf):
    for k in range(FCHUNKS):
      def body(j, acc):
        return acc + rows_v[buf, j, pl.ds(k * LANES, LANES)]
      acc = lax.fori_loop(0, L, body, jnp.zeros((LANES,), jnp.float32))
      acc_v[r, pl.ds(k * LANES, LANES)] = acc

  d0a, d0b = start_gather(0, 0)
  d0a.wait()
  d0b.wait()

  def outer(g, _):
    buf = lax.rem(g, 2)
    nxt = 1 - buf

    @pl.when(g + 1 < ROWS_PER_TILE)
    def _():
      da, db = start_gather(g + 1, nxt)

    accum_row(g, buf)

    @pl.when(g + 1 < ROWS_PER_TILE)
    def _():
      pltpu.make_async_copy(
          table_hbm.at[idx_v.at[pl.ds(0, CHUNK_A)]],
          rows_v.at[nxt, pl.ds(0, CHUNK_A)], sem_a).wait()
      pltpu.make_async_copy(
          table_hbm.at[idx_v.at[pl.ds(0, CHUNK_B)]],
          rows_v.at[nxt, pl.ds(CHUNK_A, CHUNK_B)], sem_b).wait()
    return 0

  lax.fori_loop(0, ROWS_PER_TILE, outer, 0)

  pltpu.sync_copy(acc_v, out_hbm.at[pl.ds(base, ROWS_PER_TILE)])


def _sc_pool(idx_flat, table):
  mesh = plsc.VectorSubcoreMesh(core_axis_name="c", subcore_axis_name="s")
  kern = pl.kernel(
      _sc_pool_kernel,
      out_type=jax.ShapeDtypeStruct((B, F), jnp.float32),
      mesh=mesh,
      scratch_types=[
          pltpu.VMEM((IDX_PER_TILE,), jnp.int32),
          pltpu.VMEM((2, L, F), jnp.float32),
          pltpu.VMEM((ROWS_PER_TILE, F), jnp.float32),
          pltpu.SemaphoreType.DMA,
          pltpu.SemaphoreType.DMA,
      ],
      compiler_params=pltpu.CompilerParams(use_tc_tiling_on_sc=False),
  )
  return kern(idx_flat, table)


def _tc_head_kernel(x_ref, w_ref, b_ref, bng_ref, bnb_ref, bnm_ref, bnv_ref,
                    lng_ref, lnb_ref, o_ref):
  x = x_ref[...] * (1.0 / L)
  y = jnp.dot(x, w_ref[...], preferred_element_type=jnp.float32) + b_ref[...]
  inv = lax.rsqrt(bnv_ref[...] + 1e-3)
  y = (y - bnm_ref[...]) * inv * bng_ref[...] + bnb_ref[...]
  mu = jnp.mean(y, axis=-1, keepdims=True)
  yc = y - mu
  var = jnp.mean(yc * yc, axis=-1, keepdims=True)
  o_ref[...] = yc * lax.rsqrt(var + 1e-3) * lng_ref[...] + lnb_ref[...]


def _tc_head(pooled, W, b, bn_gamma, bn_beta, bn_mean, bn_var, ln_gamma, ln_beta):
  blk = 512
  grid = B // blk
  vec_spec = pl.BlockSpec((1, F), lambda i: (0, 0))
  return pl.pallas_call(
      _tc_head_kernel,
      grid=(grid,),
      in_specs=[
          pl.BlockSpec((blk, F), lambda i: (i, 0)),
          pl.BlockSpec((F, F), lambda i: (0, 0)),
          vec_spec, vec_spec, vec_spec, vec_spec, vec_spec, vec_spec, vec_spec,
      ],
      out_specs=pl.BlockSpec((blk, F), lambda i: (i, 0)),
      out_shape=jax.ShapeDtypeStruct((B, F), jnp.float32),
  )(pooled, W, b.reshape(1, F), bn_gamma.reshape(1, F), bn_beta.reshape(1, F),
    bn_mean.reshape(1, F), bn_var.reshape(1, F), ln_gamma.reshape(1, F),
    ln_beta.reshape(1, F))


@jax.jit
def kernel(inputs, table, W, b, bn_gamma, bn_beta, bn_mean, bn_var, ln_gamma, ln_beta):
  idx_flat = inputs.astype(jnp.int32).reshape(B * L)
  pooled = _sc_pool(idx_flat, table)
  return _tc_head(pooled, W, b, bn_gamma, bn_beta, bn_mean, bn_var,
                  ln_gamma, ln_beta)


# speedup vs baseline: 1.3859x; 1.3859x over previous
import functools

import jax
import jax.numpy as jnp
from jax import lax
from jax.experimental import pallas as pl
from jax.experimental.pallas import tpu as pltpu
from jax.experimental.pallas import tpu_sc as plsc

B = 4096
L = 200
F = 64
NC = 2
NS = 16
NW = NC * NS
ROWS_PER_TILE = B // NW
IDX_PER_TILE = ROWS_PER_TILE * L
LANES = 16
FCHUNKS = F // LANES
CHUNK_A = 128
CHUNK_B = L - CHUNK_A


def _sc_pool_kernel(idx_hbm, table_hbm, out_hbm, idx_v, rows_v, acc_v, sem0, sem1):
  wid = lax.axis_index("s") * NC + lax.axis_index("c")
  base = wid * ROWS_PER_TILE

  pltpu.sync_copy(idx_hbm.at[pl.ds(base * L, IDX_PER_TILE)], idx_v)

  sems = (sem0, sem1)

  def start(r, buf):
    off = pl.multiple_of(r * L, 8)
    pltpu.async_copy(
        table_hbm.at[idx_v.at[pl.ds(off, CHUNK_A)]],
        rows_v.at[buf, pl.ds(0, CHUNK_A)], sems[buf])
    pltpu.async_copy(
        table_hbm.at[idx_v.at[pl.ds(off + CHUNK_A, CHUNK_B)]],
        rows_v.at[buf, pl.ds(CHUNK_A, CHUNK_B)], sems[buf])

  def wait(buf):
    pltpu.make_async_copy(
        table_hbm.at[idx_v.at[pl.ds(0, CHUNK_A)]],
        rows_v.at[buf, pl.ds(0, CHUNK_A)], sems[buf]).wait()
    pltpu.make_async_copy(
        table_hbm.at[idx_v.at[pl.ds(0, CHUNK_B)]],
        rows_v.at[buf, pl.ds(CHUNK_A, CHUNK_B)], sems[buf]).wait()

  def accum(r, buf):
    z = jnp.zeros((LANES,), jnp.float32)

    @plsc.parallel_loop(0, L, step=1, unroll=8, carry=(z, z, z, z))
    def accs(j, c):
      a0, a1, a2, a3 = c
      return (a0 + rows_v[buf, j, pl.ds(0 * LANES, LANES)],
              a1 + rows_v[buf, j, pl.ds(1 * LANES, LANES)],
              a2 + rows_v[buf, j, pl.ds(2 * LANES, LANES)],
              a3 + rows_v[buf, j, pl.ds(3 * LANES, LANES)])

    a0, a1, a2, a3 = accs
    acc_v[r, pl.ds(0 * LANES, LANES)] = a0
    acc_v[r, pl.ds(1 * LANES, LANES)] = a1
    acc_v[r, pl.ds(2 * LANES, LANES)] = a2
    acc_v[r, pl.ds(3 * LANES, LANES)] = a3

  start(0, 0)

  def outer(g, _):
    r0 = g * 2
    start(r0 + 1, 1)
    wait(0)
    accum(r0, 0)

    @pl.when(r0 + 2 < ROWS_PER_TILE)
    def _():
      start(r0 + 2, 0)

    wait(1)
    accum(r0 + 1, 1)
    return 0

  lax.fori_loop(0, ROWS_PER_TILE // 2, outer, 0)

  pltpu.sync_copy(acc_v, out_hbm.at[pl.ds(base, ROWS_PER_TILE)])


def _sc_pool(idx_flat, table):
  mesh = plsc.VectorSubcoreMesh(core_axis_name="c", subcore_axis_name="s")
  kern = pl.kernel(
      _sc_pool_kernel,
      out_type=jax.ShapeDtypeStruct((B, F), jnp.float32),
      mesh=mesh,
      scratch_types=[
          pltpu.VMEM((IDX_PER_TILE,), jnp.int32),
          pltpu.VMEM((2, L, F), jnp.float32),
          pltpu.VMEM((ROWS_PER_TILE, F), jnp.float32),
          pltpu.SemaphoreType.DMA,
          pltpu.SemaphoreType.DMA,
      ],
      compiler_params=pltpu.CompilerParams(use_tc_tiling_on_sc=False),
  )
  return kern(idx_flat, table)


def _tc_head_kernel(x_ref, w_ref, b_ref, bng_ref, bnb_ref, bnm_ref, bnv_ref,
                    lng_ref, lnb_ref, o_ref):
  x = x_ref[...] * (1.0 / L)
  y = jnp.dot(x, w_ref[...], preferred_element_type=jnp.float32) + b_ref[...]
  inv = lax.rsqrt(bnv_ref[...] + 1e-3)
  y = (y - bnm_ref[...]) * inv * bng_ref[...] + bnb_ref[...]
  mu = jnp.mean(y, axis=-1, keepdims=True)
  yc = y - mu
  var = jnp.mean(yc * yc, axis=-1, keepdims=True)
  o_ref[...] = yc * lax.rsqrt(var + 1e-3) * lng_ref[...] + lnb_ref[...]


def _tc_head(pooled, W, b, bn_gamma, bn_beta, bn_mean, bn_var, ln_gamma, ln_beta):
  blk = 512
  grid = B // blk
  vec_spec = pl.BlockSpec((1, F), lambda i: (0, 0))
  return pl.pallas_call(
      _tc_head_kernel,
      grid=(grid,),
      in_specs=[
          pl.BlockSpec((blk, F), lambda i: (i, 0)),
          pl.BlockSpec((F, F), lambda i: (0, 0)),
          vec_spec, vec_spec, vec_spec, vec_spec, vec_spec, vec_spec, vec_spec,
      ],
      out_specs=pl.BlockSpec((blk, F), lambda i: (i, 0)),
      out_shape=jax.ShapeDtypeStruct((B, F), jnp.float32),
  )(pooled, W, b.reshape(1, F), bn_gamma.reshape(1, F), bn_beta.reshape(1, F),
    bn_mean.reshape(1, F), bn_var.reshape(1, F), ln_gamma.reshape(1, F),
    ln_beta.reshape(1, F))


@jax.jit
def kernel(inputs, table, W, b, bn_gamma, bn_beta, bn_mean, bn_var, ln_gamma, ln_beta):
  idx_flat = inputs.astype(jnp.int32).reshape(B * L)
  pooled = _sc_pool(idx_flat, table)
  return _tc_head(pooled, W, b, bn_gamma, bn_beta, bn_mean, bn_var,
                  ln_gamma, ln_beta)
